# Initial kernel scaffold; baseline (speedup 1.0000x reference)
import jax
import jax.numpy as jnp
from jax.experimental import pallas as pl


def kernel(x, edge_index, W1_l, b1, W1_r, W2_l, b2, W2_r):
    raise NotImplementedError("write your pallas kernel here")


# speedup vs baseline: 9.7941x; 9.7941x over previous
import functools

import jax
import jax.numpy as jnp
from jax import lax
from jax.experimental import pallas as pl
from jax.experimental.pallas import tpu as pltpu
from jax.experimental.pallas import tpu_sc as plsc

N = 10000
D = 128
NT = 32
EPT = 10000
NCH = 125
ECH = 80
RPT = N // 16
DEGW = 128

_MESH = plsc.VectorSubcoreMesh(core_axis_name="c", subcore_axis_name="s")


@functools.partial(
    pl.kernel,
    mesh=_MESH,
    out_type=jax.ShapeDtypeStruct((2, 16, RPT, DEGW), jnp.float32),
    scratch_types=[
        pltpu.VMEM((NCH, ECH), jnp.int32),
        pltpu.VMEM((ECH, DEGW), jnp.float32),
        pltpu.VMEM((25, DEGW), jnp.float32),
        pltpu.VMEM_SHARED((N, DEGW), jnp.float32),
    ],
)
def _deg_sc(dst_hbm, out_hbm, dst_v, ones_v, zbuf, acc):
    c = lax.axis_index("c")
    s = lax.axis_index("s")
    tid = s * 2 + c
    one16 = jnp.ones((16,), jnp.float32)
    zero16 = jnp.zeros((16,), jnp.float32)

    def fill(i, _):
        for k in range(DEGW // 16):
            ones_v[i, pl.ds(k * 16, 16)] = one16
        return 0

    lax.fori_loop(0, ECH, fill, 0)

    def zrow(i, _):
        for k in range(DEGW // 16):
            zbuf[i, pl.ds(k * 16, 16)] = zero16
        return 0

    lax.fori_loop(0, 25, zrow, 0)
    for j in range(RPT // 25):
        pltpu.sync_copy(zbuf, acc.at[pl.ds(s * RPT + j * 25, 25)])
    pltpu.sync_copy(dst_hbm.at[tid], dst_v)
    plsc.subcore_barrier()

    def chunk(ci, _):
        pltpu.sync_copy(ones_v, acc.at[dst_v.at[ci]], add=True)
        return 0

    lax.fori_loop(0, NCH, chunk, 0)
    plsc.subcore_barrier()
    pltpu.sync_copy(acc.at[pl.ds(s * RPT, RPT)], out_hbm.at[c].at[s])


@functools.partial(
    pl.kernel,
    mesh=_MESH,
    out_type=jax.ShapeDtypeStruct((2, 16, RPT, D), jnp.float32),
    scratch_types=[
        pltpu.VMEM((NCH, ECH), jnp.int32),
        pltpu.VMEM((NCH, ECH), jnp.int32),
        pltpu.VMEM((ECH, D), jnp.float32),
        pltpu.VMEM((25, D), jnp.float32),
        pltpu.VMEM_SHARED((N, D), jnp.float32),
        pltpu.SemaphoreType.DMA,
    ],
)
def _spmm_sc(src_hbm, dst_hbm, x_hbm, out_hbm, src_v, dst_v, rows, zbuf, acc,
             gsem):
    c = lax.axis_index("c")
    s = lax.axis_index("s")
    tid = s * 2 + c
    zero16 = jnp.zeros((16,), jnp.float32)

    def zrow(i, _):
        for k in range(D // 16):
            zbuf[i, pl.ds(k * 16, 16)] = zero16
        return 0

    lax.fori_loop(0, 25, zrow, 0)
    for j in range(RPT // 25):
        pltpu.sync_copy(zbuf, acc.at[pl.ds(s * RPT + j * 25, 25)])
    pltpu.sync_copy(src_hbm.at[tid], src_v)
    pltpu.sync_copy(dst_hbm.at[tid], dst_v)
    plsc.subcore_barrier()

    def chunk(ci, _):
        pltpu.async_copy(x_hbm.at[src_v.at[ci]], rows, gsem).wait()
        pltpu.sync_copy(rows, acc.at[dst_v.at[ci]], add=True)
        return 0

    lax.fori_loop(0, NCH, chunk, 0)
    plsc.subcore_barrier()
    pltpu.sync_copy(acc.at[pl.ds(s * RPT, RPT)], out_hbm.at[c].at[s])


R = 1000
_SELU_ALPHA = 1.6732632423543772
_SELU_SCALE = 1.0507009873554805


def _dinv_of(deg_ref):
    dg = deg_ref[0, :, 0:1] + deg_ref[1, :, 0:1]
    return jnp.where(dg > 0.0, lax.rsqrt(dg), 0.0)


def _dninv_of(deg_ref):
    dg = deg_ref[0, :, 0:1] + deg_ref[1, :, 0:1]
    return 1.0 / jnp.maximum(dg, 1.0)


def _matt(a, w):
    return lax.dot_general(a, w, (((1,), (1,)), ((), ())),
                           preferred_element_type=jnp.float32)


_deg_spec = pl.BlockSpec((2, R, DEGW), lambda i: (0, i, 0))
_p_spec = pl.BlockSpec((2, R, D), lambda i: (0, i, 0))
_m_spec = pl.BlockSpec((R, D), lambda i: (i, 0))
_w_spec = pl.BlockSpec((D, D), lambda i: (0, 0))
_b_spec = pl.BlockSpec((1, D), lambda i: (0, 0))
_m_shape = jax.ShapeDtypeStruct((N, D), jnp.float32)


def _t0_body(deg_ref, x_ref, g0_ref):
    g0_ref[...] = x_ref[...] * _dinv_of(deg_ref)


_t0 = pl.pallas_call(
    _t0_body, grid=(N // R,),
    in_specs=[_deg_spec, _m_spec],
    out_specs=_m_spec, out_shape=_m_shape)


def _t1_body(deg_ref, p_ref, x_ref, h1_ref, g1_ref):
    dinv = _dinv_of(deg_ref)
    h1 = (p_ref[0] + p_ref[1]) * dinv + x_ref[...]
    h1_ref[...] = h1
    g1_ref[...] = h1 * dinv


_t1 = pl.pallas_call(
    _t1_body, grid=(N // R,),
    in_specs=[_deg_spec, _p_spec, _m_spec],
    out_specs=(_m_spec, _m_spec), out_shape=(_m_shape, _m_shape))


def _t2_body(deg_ref, p_ref, h1_ref, h_ref):
    h_ref[...] = (p_ref[0] + p_ref[1]) * _dinv_of(deg_ref) + h1_ref[...]


_t2 = pl.pallas_call(
    _t2_body, grid=(N // R,),
    in_specs=[_deg_spec, _p_spec, _m_spec],
    out_specs=_m_spec, out_shape=_m_shape)


def _t3_body(deg_ref, p_ref, h_ref, w1l_ref, b1_ref, w1r_ref, h2_ref):
    mean1 = (p_ref[0] + p_ref[1]) * _dninv_of(deg_ref)
    z = _matt(mean1, w1l_ref[...]) + b1_ref[...] + _matt(h_ref[...],
                                                         w1r_ref[...])
    h2_ref[...] = _SELU_SCALE * jnp.where(
        z > 0.0, z, _SELU_ALPHA * (jnp.exp(z) - 1.0))


_t3 = pl.pallas_call(
    _t3_body, grid=(N // R,),
    in_specs=[_deg_spec, _p_spec, _m_spec, _w_spec, _b_spec, _w_spec],
    out_specs=_m_spec, out_shape=_m_shape)


def _t4_body(deg_ref, p_ref, h2_ref, w2l_ref, b2_ref, w2r_ref, out_ref):
    mean2 = (p_ref[0] + p_ref[1]) * _dninv_of(deg_ref)
    z = _matt(mean2, w2l_ref[...]) + b2_ref[...] + _matt(h2_ref[...],
                                                         w2r_ref[...])
    z = z - jnp.max(z, axis=1, keepdims=True)
    ez = jnp.exp(z)
    out_ref[...] = ez / jnp.sum(ez, axis=1, keepdims=True)


_t4 = pl.pallas_call(
    _t4_body, grid=(N // R,),
    in_specs=[_deg_spec, _p_spec, _m_spec, _w_spec, _b_spec, _w_spec],
    out_specs=_m_spec, out_shape=_m_shape)


def kernel(x, edge_index, W1_l, b1, W1_r, W2_l, b2, W2_r):
    src = edge_index[0].reshape(NT, NCH, ECH)
    dst = edge_index[1].reshape(NT, NCH, ECH)
    b1r = b1.reshape(1, D)
    b2r = b2.reshape(1, D)

    def spmm(xin):
        return _spmm_sc(src, dst, xin).reshape(2, N, D)

    deg16 = _deg_sc(dst).reshape(2, N, DEGW)
    g0 = _t0(deg16, x)
    p1 = spmm(g0)
    h1, g1 = _t1(deg16, p1, x)
    p2 = spmm(g1)
    h = _t2(deg16, p2, h1)
    p3 = spmm(h)
    h2 = _t3(deg16, p3, h, W1_l, b1r, W1_r)
    p4 = spmm(h2)
    out = _t4(deg16, p4, h2, W2_l, b2r, W2_r)
    return out


# speedup vs baseline: 14.8860x; 1.5199x over previous
import functools

import jax
import jax.numpy as jnp
from jax import lax
from jax.experimental import pallas as pl
from jax.experimental.pallas import tpu as pltpu
from jax.experimental.pallas import tpu_sc as plsc

N = 10000
D = 128
NT = 32
EPT = 10000
NCH = 125
ECH = 80

RPT = N // 16
DEGW = 128

_MESH = plsc.VectorSubcoreMesh(core_axis_name="c", subcore_axis_name="s")


@functools.partial(
    pl.kernel,
    mesh=_MESH,
    out_type=jax.ShapeDtypeStruct((2, 16, RPT, DEGW), jnp.float32),
    scratch_types=[
        pltpu.VMEM((NCH, ECH), jnp.int32),
        pltpu.VMEM((ECH, DEGW), jnp.float32),
        pltpu.VMEM((25, DEGW), jnp.float32),
        pltpu.VMEM_SHARED((N, DEGW), jnp.float32),
    ],
)
def _deg_sc(dst_hbm, out_hbm, dst_v, ones_v, zbuf, acc):
    c = lax.axis_index("c")
    s = lax.axis_index("s")
    tid = s * 2 + c
    one16 = jnp.ones((16,), jnp.float32)
    zero16 = jnp.zeros((16,), jnp.float32)

    def fill(i, _):
        for k in range(DEGW // 16):
            ones_v[i, pl.ds(k * 16, 16)] = one16
        return 0

    lax.fori_loop(0, ECH, fill, 0)

    def zrow(i, _):
        for k in range(DEGW // 16):
            zbuf[i, pl.ds(k * 16, 16)] = zero16
        return 0

    lax.fori_loop(0, 25, zrow, 0)
    for j in range(RPT // 25):
        pltpu.sync_copy(zbuf, acc.at[pl.ds(s * RPT + j * 25, 25)])
    pltpu.sync_copy(dst_hbm.at[tid], dst_v)
    plsc.subcore_barrier()

    def chunk(ci, _):
        pltpu.sync_copy(ones_v, acc.at[dst_v.at[ci]], add=True)
        return 0

    lax.fori_loop(0, NCH, chunk, 0)
    plsc.subcore_barrier()
    pltpu.sync_copy(acc.at[pl.ds(s * RPT, RPT)], out_hbm.at[c].at[s])


@functools.partial(
    pl.kernel,
    mesh=_MESH,
    out_type=jax.ShapeDtypeStruct((2, 16, RPT, D), jnp.float32),
    scratch_types=[
        pltpu.VMEM((NCH, ECH), jnp.int32),
        pltpu.VMEM((ECH,), jnp.int32),
        pltpu.VMEM((ECH,), jnp.int32),
        pltpu.VMEM((ECH,), jnp.int32),
        pltpu.VMEM((ECH,), jnp.int32),
        pltpu.VMEM((ECH, D), jnp.float32),
        pltpu.VMEM((ECH, D), jnp.float32),
        pltpu.VMEM((25, D), jnp.float32),
        pltpu.VMEM_SHARED((N, D), jnp.float32),
        pltpu.SemaphoreType.DMA,
        pltpu.SemaphoreType.DMA,
        pltpu.SemaphoreType.DMA,
        pltpu.SemaphoreType.DMA,
    ],
)
def _spmm_sc(sd_hbm, x_hbm, out_hbm, sd_v, s0, d0, s1, d1, rows0, rows1,
             zbuf, acc, ga, gb, sa, sb):
    c = lax.axis_index("c")
    s = lax.axis_index("s")
    tid = s * 2 + c
    zero16 = jnp.zeros((16,), jnp.float32)

    def zrow(i, _):
        for k in range(D // 16):
            zbuf[i, pl.ds(k * 16, 16)] = zero16
        return 0

    lax.fori_loop(0, 25, zrow, 0)
    for j in range(RPT // 25):
        pltpu.sync_copy(zbuf, acc.at[pl.ds(s * RPT + j * 25, 25)])
    pltpu.sync_copy(sd_hbm.at[tid], sd_v)
    plsc.subcore_barrier()

    def unpack(ci, sbuf, dbuf):
        for k in range(ECH // 16):
            v = sd_v[ci, pl.ds(k * 16, 16)]
            sbuf[pl.ds(k * 16, 16)] = jnp.bitwise_and(v, 0xFFFF)
            dbuf[pl.ds(k * 16, 16)] = lax.shift_right_logical(v, 16)

    def gstart(sbuf, buf, sem):
        pltpu.async_copy(x_hbm.at[sbuf], buf, sem)

    def gwait(buf, sem):
        pltpu.make_async_copy(x_hbm.at[s0], buf, sem).wait()

    def sstart(dbuf, buf, sem):
        pltpu.async_copy(buf, acc.at[dbuf], sem, add=True)

    def swait(buf, sem):
        pltpu.make_async_copy(buf, acc.at[d0], sem).wait()

    NPAIR = NCH // 2
    unpack(0, s0, d0)
    gstart(s0, rows0, ga)

    def pair(i, _):
        ci = 2 * i

        @pl.when(i > 0)
        def _():
            swait(rows1, sb)

        unpack(ci + 1, s1, d1)
        gstart(s1, rows1, gb)
        gwait(rows0, ga)
        sstart(d0, rows0, sa)

        @pl.when(i < NPAIR - 1)
        def _():
            swait(rows0, sa)
            unpack(ci + 2, s0, d0)
            gstart(s0, rows0, ga)

        gwait(rows1, gb)
        sstart(d1, rows1, sb)
        return 0

    lax.fori_loop(0, NPAIR, pair, 0)
    swait(rows0, sa)
    unpack(NCH - 1, s0, d0)
    gstart(s0, rows0, ga)
    gwait(rows0, ga)
    sstart(d0, rows0, sa)
    swait(rows0, sa)
    swait(rows1, sb)
    plsc.subcore_barrier()
    pltpu.sync_copy(acc.at[pl.ds(s * RPT, RPT)], out_hbm.at[c].at[s])


R = 1000
_SELU_ALPHA = 1.6732632423543772
_SELU_SCALE = 1.0507009873554805


def _dinv_of(deg_ref):
    dg = deg_ref[0, :, 0:1] + deg_ref[1, :, 0:1]
    return jnp.where(dg > 0.0, lax.rsqrt(dg), 0.0)


def _dninv_of(deg_ref):
    dg = deg_ref[0, :, 0:1] + deg_ref[1, :, 0:1]
    return 1.0 / jnp.maximum(dg, 1.0)


def _matt(a, w):
    return lax.dot_general(a, w, (((1,), (1,)), ((), ())),
                           preferred_element_type=jnp.float32)


_deg_spec = pl.BlockSpec((2, R, DEGW), lambda i: (0, i, 0))
_p_spec = pl.BlockSpec((2, R, D), lambda i: (0, i, 0))
_m_spec = pl.BlockSpec((R, D), lambda i: (i, 0))
_w_spec = pl.BlockSpec((D, D), lambda i: (0, 0))
_b_spec = pl.BlockSpec((1, D), lambda i: (0, 0))
_m_shape = jax.ShapeDtypeStruct((N, D), jnp.float32)


def _t0_body(deg_ref, x_ref, g0_ref):
    g0_ref[...] = x_ref[...] * _dinv_of(deg_ref)


_t0 = pl.pallas_call(
    _t0_body, grid=(N // R,),
    in_specs=[_deg_spec, _m_spec],
    out_specs=_m_spec, out_shape=_m_shape)


def _t1_body(deg_ref, p_ref, x_ref, h1_ref, g1_ref):
    dinv = _dinv_of(deg_ref)
    h1 = (p_ref[0] + p_ref[1]) * dinv + x_ref[...]
    h1_ref[...] = h1
    g1_ref[...] = h1 * dinv


_t1 = pl.pallas_call(
    _t1_body, grid=(N // R,),
    in_specs=[_deg_spec, _p_spec, _m_spec],
    out_specs=(_m_spec, _m_spec), out_shape=(_m_shape, _m_shape))


def _t2_body(deg_ref, p_ref, h1_ref, h_ref):
    h_ref[...] = (p_ref[0] + p_ref[1]) * _dinv_of(deg_ref) + h1_ref[...]


_t2 = pl.pallas_call(
    _t2_body, grid=(N // R,),
    in_specs=[_deg_spec, _p_spec, _m_spec],
    out_specs=_m_spec, out_shape=_m_shape)


def _t3_body(deg_ref, p_ref, h_ref, w1l_ref, b1_ref, w1r_ref, h2_ref):
    mean1 = (p_ref[0] + p_ref[1]) * _dninv_of(deg_ref)
    z = _matt(mean1, w1l_ref[...]) + b1_ref[...] + _matt(h_ref[...],
                                                         w1r_ref[...])
    h2_ref[...] = _SELU_SCALE * jnp.where(
        z > 0.0, z, _SELU_ALPHA * (jnp.exp(z) - 1.0))


_t3 = pl.pallas_call(
    _t3_body, grid=(N // R,),
    in_specs=[_deg_spec, _p_spec, _m_spec, _w_spec, _b_spec, _w_spec],
    out_specs=_m_spec, out_shape=_m_shape)


def _t4_body(deg_ref, p_ref, h2_ref, w2l_ref, b2_ref, w2r_ref, out_ref):
    mean2 = (p_ref[0] + p_ref[1]) * _dninv_of(deg_ref)
    z = _matt(mean2, w2l_ref[...]) + b2_ref[...] + _matt(h2_ref[...],
                                                         w2r_ref[...])
    z = z - jnp.max(z, axis=1, keepdims=True)
    ez = jnp.exp(z)
    out_ref[...] = ez / jnp.sum(ez, axis=1, keepdims=True)


_t4 = pl.pallas_call(
    _t4_body, grid=(N // R,),
    in_specs=[_deg_spec, _p_spec, _m_spec, _w_spec, _b_spec, _w_spec],
    out_specs=_m_spec, out_shape=_m_shape)


def kernel(x, edge_index, W1_l, b1, W1_r, W2_l, b2, W2_r):
    sd = (edge_index[0] | (edge_index[1] << 16)).reshape(NT, NCH, ECH)
    dstd = edge_index[1].reshape(NT, NCH, ECH)
    b1r = b1.reshape(1, D)
    b2r = b2.reshape(1, D)

    def spmm(xin):
        return _spmm_sc(sd, xin).reshape(2, N, D)

    deg16 = _deg_sc(dstd).reshape(2, N, DEGW)
    g0 = _t0(deg16, x)
    p1 = spmm(g0)
    h1, g1 = _t1(deg16, p1, x)
    p2 = spmm(g1)
    h = _t2(deg16, p2, h1)
    p3 = spmm(h)
    h2 = _t3(deg16, p3, h, W1_l, b1r, W1_r)
    p4 = spmm(h2)
    out = _t4(deg16, p4, h2, W2_l, b2r, W2_r)
    return out
